# baseline (speedup 1.0000x reference)
import functools

import jax
import jax.numpy as jnp
import numpy as np
from jax import lax
from jax.experimental import pallas as pl
from jax.experimental.pallas import tpu as pltpu

B, LEN, CH = 4, 4096, 1024
G = 4
LO = LEN // G
LBLK = 128


def _tc_body(in_ref, out_ref):
    for hi in range(G):
        out_ref[0, hi] = in_ref[0, :, hi, :]


_tc_shuffle = pl.pallas_call(
    _tc_body,
    grid=(B, LO // LBLK),
    in_specs=[
        pl.BlockSpec((1, LBLK, G, CH), lambda b, l: (b, l, 0, 0)),
    ],
    out_specs=pl.BlockSpec((1, G, LBLK, CH), lambda b, l: (b, 0, l, 0)),
    out_shape=jax.ShapeDtypeStruct((B, G, LO, CH), jnp.float32),
)


def kernel(inputs):
    in4 = inputs.reshape(B, LO, G, CH)
    out4 = _tc_shuffle(in4)
    return out4.reshape(B, LEN, CH)


# speedup vs baseline: 2.3739x; 2.3739x over previous
import functools

import jax
import jax.numpy as jnp
import numpy as np
from jax import lax
from jax.experimental import pallas as pl
from jax.experimental.pallas import tpu as pltpu

B, LEN, CH = 4, 4096, 1024
G = 4
LO = LEN // G
LBLK = 128


def _tc_body(in_ref, out_ref):
    x = in_ref[0].reshape(LBLK, G, CH)
    out_ref[0] = jnp.swapaxes(x, 0, 1)


_tc_shuffle = pl.pallas_call(
    _tc_body,
    grid=(B, LO // LBLK),
    in_specs=[
        pl.BlockSpec((1, G * LBLK, CH), lambda b, l: (b, l, 0)),
    ],
    out_specs=pl.BlockSpec((1, G, LBLK, CH), lambda b, l: (b, 0, l, 0)),
    out_shape=jax.ShapeDtypeStruct((B, G, LO, CH), jnp.float32),
)


def kernel(inputs):
    out4 = _tc_shuffle(inputs)
    return out4.reshape(B, LEN, CH)
